# baseline (speedup 1.0000x reference)
import functools

import jax
import jax.numpy as jnp
from jax import lax
from jax.experimental import pallas as pl
from jax.experimental.pallas import tpu as pltpu, tpu_sc as plsc

B = 4 * 8192
D = 4096
NC, NS = 2, 16
NW = NC * NS
BPW = B // NW
L = 16
G = BPW // L


def _encoder_body(x_hbm, w_hbm, out_hbm, idx_v, w_v, drain_v, wsem):
    wid = lax.axis_index("s") * NC + lax.axis_index("c")
    base = wid * BPW

    pltpu.sync_copy(x_hbm.at[pl.ds(base, BPW)], idx_v)
    pltpu.sync_copy(w_hbm, w_v)

    def group(g, carry):
        a = idx_v[pl.ds(g * L, L)]
        row = base + g * L
        for l in range(L):
            pltpu.async_copy(w_v.at[a[l]], out_hbm.at[row + l], wsem)

        @pl.when(g >= 2)
        def _():
            pltpu.make_async_copy(
                out_hbm.at[pl.ds(base, L)], drain_v, wsem
            ).wait()

        return carry

    lax.fori_loop(0, G, group, 0, unroll=False)

    pltpu.make_async_copy(out_hbm.at[pl.ds(base, L)], drain_v, wsem).wait()
    pltpu.make_async_copy(out_hbm.at[pl.ds(base, L)], drain_v, wsem).wait()


@functools.partial(jax.jit, static_argnames=())
def kernel(x, emb_weight):
    mesh = plsc.VectorSubcoreMesh(core_axis_name="c", subcore_axis_name="s")
    run = pl.kernel(
        _encoder_body,
        out_type=jax.ShapeDtypeStruct((B, D), jnp.float32),
        mesh=mesh,
        scratch_types=[
            pltpu.VMEM((BPW,), jnp.int32),
            pltpu.VMEM((2, D), jnp.float32),
            pltpu.VMEM((L, D), jnp.float32),
            pltpu.SemaphoreType.DMA,
        ],
    )
    out = run(x.reshape(B).astype(jnp.int32), emb_weight)
    return out.reshape(x.shape + (D,))


# speedup vs baseline: 1.0012x; 1.0012x over previous
import functools

import jax
import jax.numpy as jnp
from jax import lax
from jax.experimental import pallas as pl
from jax.experimental.pallas import tpu as pltpu, tpu_sc as plsc

B = 4 * 8192
D = 4096
NC, NS = 2, 16
NW = NC * NS
BPW = B // NW
L = 16
G = BPW // L


def _encoder_body(x_hbm, w_hbm, out_hbm, idx_v, w_v, drain_v, wsem):
    wid = lax.axis_index("s") * NC + lax.axis_index("c")
    base = wid * BPW

    pltpu.sync_copy(x_hbm.at[pl.ds(base, BPW)], idx_v)
    pltpu.sync_copy(w_hbm, w_v)

    def group(g, a):
        row = base + g * L
        for l in range(L):
            pltpu.async_copy(w_v.at[a[l]], out_hbm.at[row + l], wsem)
        a_next = idx_v[pl.ds((g + 1) * L - (g == G - 1) * L, L)]

        @pl.when(g >= 2)
        def _():
            pltpu.make_async_copy(
                out_hbm.at[pl.ds(base, L)], drain_v, wsem
            ).wait()

        return a_next

    lax.fori_loop(0, G, group, idx_v[pl.ds(0, L)], unroll=False)

    pltpu.make_async_copy(out_hbm.at[pl.ds(base, L)], drain_v, wsem).wait()
    pltpu.make_async_copy(out_hbm.at[pl.ds(base, L)], drain_v, wsem).wait()


@functools.partial(jax.jit, static_argnames=())
def kernel(x, emb_weight):
    mesh = plsc.VectorSubcoreMesh(core_axis_name="c", subcore_axis_name="s")
    run = pl.kernel(
        _encoder_body,
        out_type=jax.ShapeDtypeStruct((B, D), jnp.float32),
        mesh=mesh,
        scratch_types=[
            pltpu.VMEM((BPW,), jnp.int32),
            pltpu.VMEM((2, D), jnp.float32),
            pltpu.VMEM((L, D), jnp.float32),
            pltpu.SemaphoreType.DMA,
        ],
    )
    out = run(x.reshape(B).astype(jnp.int32), emb_weight)
    return out.reshape(x.shape + (D,))


# speedup vs baseline: 1.0034x; 1.0022x over previous
import functools

import jax
import jax.numpy as jnp
from jax import lax
from jax.experimental import pallas as pl
from jax.experimental.pallas import tpu as pltpu, tpu_sc as plsc

B = 4 * 8192
D = 4096
NC, NS = 2, 16
NW = NC * NS
BPW = B // NW
L = 16
G = BPW // L


def _encoder_body(x_hbm, w_hbm, out_hbm, idx_v, w_v, drain_v, wsem):
    wid = lax.axis_index("s") * NC + lax.axis_index("c")
    base = wid * BPW

    pltpu.sync_copy(x_hbm.at[pl.ds(base, BPW)], idx_v)
    pltpu.sync_copy(w_hbm, w_v)

    def group(g, carry):
        a = idx_v[pl.ds(g * L, L)]
        row = base + g * L
        for l in range(L):
            pltpu.async_copy(w_v.at[a[l]], out_hbm.at[row + l], wsem)

        @pl.when(g >= 1)
        def _():
            pltpu.make_async_copy(
                out_hbm.at[pl.ds(base, L)], drain_v, wsem
            ).wait()

        return carry

    lax.fori_loop(0, G, group, 0, unroll=False)

    pltpu.make_async_copy(out_hbm.at[pl.ds(base, L)], drain_v, wsem).wait()


@functools.partial(jax.jit, static_argnames=())
def kernel(x, emb_weight):
    mesh = plsc.VectorSubcoreMesh(core_axis_name="c", subcore_axis_name="s")
    run = pl.kernel(
        _encoder_body,
        out_type=jax.ShapeDtypeStruct((B, D), jnp.float32),
        mesh=mesh,
        scratch_types=[
            pltpu.VMEM((BPW,), jnp.int32),
            pltpu.VMEM((2, D), jnp.float32),
            pltpu.VMEM((L, D), jnp.float32),
            pltpu.SemaphoreType.DMA,
        ],
    )
    out = run(x.reshape(B).astype(jnp.int32), emb_weight)
    return out.reshape(x.shape + (D,))
